# baseline (speedup 1.0000x reference)
import functools

import jax
import jax.numpy as jnp
from jax import lax
from jax.experimental import pallas as pl
from jax.experimental.pallas import tpu as pltpu
from jax.experimental.pallas import tpu_sc as plsc

NC = 2
NS = 16
CH = 128
NBUF = 1
SB = 16
NBLK = 5


def _make_agg_kernel(n, d, e):
    nw = NC * NS
    assert e == nw * NBLK * SB * CH
    rows_per_tile = (n // NS) // 8 * 8
    tail_rows = n - rows_per_tile * NS
    tail_base = rows_per_tile * NS
    assert tail_base % 8 == 0 and tail_rows % 8 == 0

    mesh = plsc.VectorSubcoreMesh(core_axis_name="c", subcore_axis_name="s")

    @functools.partial(
        pl.kernel,
        out_type=jax.ShapeDtypeStruct((NC, n, d), jnp.float32),
        mesh=mesh,
        scratch_types=[
            pltpu.VMEM((SB, CH), jnp.int32),
            pltpu.VMEM((SB, CH), jnp.int32),
            pltpu.VMEM((NBUF, CH, d), jnp.float32),
            pltpu.VMEM_SHARED((n + 8, d), jnp.float32),
            pltpu.SemaphoreType.DMA((NBUF,)),
        ],
    )
    def agg(x_hbm, src_hbm, dst_hbm, out_hbm, src_v, dst_v, rows_v, acc_sh, sem):
        c = lax.axis_index("c")
        s = lax.axis_index("s")
        wid = s * NC + c
        row0 = s * rows_per_tile
        pltpu.sync_copy(x_hbm.at[pl.ds(row0, rows_per_tile)],
                        acc_sh.at[pl.ds(row0, rows_per_tile)])
        if tail_rows:
            @pl.when(s == 0)
            def _seed_tail():
                pltpu.sync_copy(x_hbm.at[pl.ds(tail_base, tail_rows)],
                                acc_sh.at[pl.ds(tail_base, tail_rows)])
        plsc.subcore_barrier()

        def block(blk, carry):
            pltpu.sync_copy(src_hbm.at[wid, blk], src_v)
            pltpu.sync_copy(dst_hbm.at[wid, blk], dst_v)

            def body(i, carry2):
                pltpu.async_copy(x_hbm.at[src_v.at[i]], rows_v.at[0],
                                 sem.at[0]).wait()
                pltpu.sync_copy(rows_v.at[0], acc_sh.at[dst_v.at[i]],
                                add=True)
                return carry2

            lax.fori_loop(0, SB, body, 0)
            return carry

        lax.fori_loop(0, NBLK, block, 0)
        plsc.subcore_barrier()
        pltpu.sync_copy(acc_sh.at[pl.ds(row0, rows_per_tile)],
                        out_hbm.at[c, pl.ds(row0, rows_per_tile)])
        if tail_rows:
            @pl.when(s == 0)
            def _write_tail():
                pltpu.sync_copy(acc_sh.at[pl.ds(tail_base, tail_rows)],
                                out_hbm.at[c, pl.ds(tail_base, tail_rows)])

    return agg


def _dense_layer(h0, W1, b1, g1, be1, W2, b2, g, be):
    def bn(h, gg, bb):
        m = jnp.mean(h, axis=0, keepdims=True)
        v = jnp.mean(h * h, axis=0, keepdims=True) - m * m
        return (h - m) * lax.rsqrt(v + 1e-5) * gg + bb

    h = jnp.dot(h0, W1, preferred_element_type=jnp.float32) + b1
    h = jnp.maximum(bn(h, g1, be1), 0.0)
    h = jnp.dot(h, W2, preferred_element_type=jnp.float32) + b2
    return jnp.maximum(bn(h, g, be), 0.0)


def _tc_layer0_body(x_ref, p_ref, W1_ref, b1_ref, g1_ref, be1_ref,
                    W2_ref, b2_ref, g_ref, be_ref, out_ref):
    h0 = p_ref[0] + p_ref[1] - x_ref[...]
    out_ref[...] = _dense_layer(h0, W1_ref[...], b1_ref[...], g1_ref[...],
                                be1_ref[...], W2_ref[...], b2_ref[...],
                                g_ref[...], be_ref[...])


def _tc_final_body(h1_ref, p_ref, batch_ref, W1_ref, b1_ref, g1_ref, be1_ref,
                   W2_ref, b2_ref, g_ref, be_ref, fc1W_ref, fc1b_ref,
                   fc2W_ref, fc2b_ref, out_ref, *, num_graphs):
    h1 = h1_ref[...]
    h0 = p_ref[0] + p_ref[1] - h1
    h2 = _dense_layer(h0, W1_ref[...], b1_ref[...], g1_ref[...], be1_ref[...],
                      W2_ref[...], b2_ref[...], g_ref[...], be_ref[...])
    n = h1.shape[0]
    gids = lax.broadcasted_iota(jnp.int32, (num_graphs, n), 0)
    onehot = (batch_ref[...] == gids).astype(jnp.float32)
    pooled0 = jnp.dot(onehot, h1, preferred_element_type=jnp.float32)
    pooled1 = jnp.dot(onehot, h2, preferred_element_type=jnp.float32)
    cat = jnp.concatenate([pooled0, pooled1], axis=-1)
    o = jnp.maximum(jnp.dot(cat, fc1W_ref[...],
                            preferred_element_type=jnp.float32) + fc1b_ref[...], 0.0)
    out_ref[...] = jnp.dot(o, fc2W_ref[...],
                           preferred_element_type=jnp.float32) + fc2b_ref[...]


def kernel(x, edge_index, batch,
           W1_0, b1_0, g1_0, be1_0, W2_0, b2_0, g_0, be_0,
           W1_1, b1_1, g1_1, be1_1, W2_1, b2_1, g_1, be_1,
           fc1_W, fc1_b, fc2_W, fc2_b):
    n, d = x.shape
    e = edge_index.shape[1]
    h_dim = W1_0.shape[1]
    num_graphs = 64
    c_dim = fc2_W.shape[1]

    nw = NC * NS
    e_pad = nw * NBLK * SB * CH
    src_p = jnp.concatenate([edge_index[0],
                             jnp.zeros((e_pad - e,), jnp.int32)])
    dst_p = jnp.concatenate([edge_index[1],
                             jnp.full((e_pad - e,), n, jnp.int32)])
    src2 = src_p.reshape(nw, NBLK, SB, CH)
    dst2 = dst_p.reshape(nw, NBLK, SB, CH)
    batch2 = batch.reshape(1, n)

    agg_fn = _make_agg_kernel(n, d, e_pad)

    p0 = agg_fn(x, src2, dst2)
    h1 = pl.pallas_call(
        _tc_layer0_body,
        out_shape=jax.ShapeDtypeStruct((n, h_dim), jnp.float32),
    )(x, p0, W1_0, b1_0.reshape(1, -1), g1_0.reshape(1, -1),
      be1_0.reshape(1, -1), W2_0, b2_0.reshape(1, -1), g_0.reshape(1, -1),
      be_0.reshape(1, -1))

    p1 = agg_fn(h1, src2, dst2)
    out = pl.pallas_call(
        functools.partial(_tc_final_body, num_graphs=num_graphs),
        out_shape=jax.ShapeDtypeStruct((num_graphs, c_dim), jnp.float32),
    )(h1, p1, batch2, W1_1, b1_1.reshape(1, -1), g1_1.reshape(1, -1),
      be1_1.reshape(1, -1), W2_1, b2_1.reshape(1, -1), g_1.reshape(1, -1),
      be_1.reshape(1, -1), fc1_W, fc1_b.reshape(1, -1), fc2_W,
      fc2_b.reshape(1, -1))
    return out


# speedup vs baseline: 4.1074x; 4.1074x over previous
import functools

import jax
import jax.numpy as jnp
from jax import lax
from jax.experimental import pallas as pl
from jax.experimental.pallas import tpu as pltpu
from jax.experimental.pallas import tpu_sc as plsc

NC = 2
NS = 16
CH = 80
NBUF = 4
SB = 25
NBLK = 5


def _make_agg_kernel(n, d, e):
    nw = NC * NS
    assert e == nw * NBLK * SB * CH
    rows_per_tile = (n // NS) // 8 * 8
    tail_rows = n - rows_per_tile * NS
    tail_base = rows_per_tile * NS
    assert tail_base % 8 == 0 and tail_rows % 8 == 0

    mesh = plsc.VectorSubcoreMesh(core_axis_name="c", subcore_axis_name="s")

    @functools.partial(
        pl.kernel,
        out_type=jax.ShapeDtypeStruct((NC, n, d), jnp.float32),
        mesh=mesh,
        scratch_types=[
            pltpu.VMEM((SB, CH), jnp.int32),
            pltpu.VMEM((SB, CH), jnp.int32),
            pltpu.VMEM((NBUF, CH, d), jnp.float32),
            pltpu.VMEM_SHARED((n, d), jnp.float32),
            pltpu.SemaphoreType.DMA((NBUF,)),
        ],
    )
    def agg(x_hbm, src_hbm, dst_hbm, out_hbm, src_v, dst_v, rows_v, acc_sh, sem):
        c = lax.axis_index("c")
        s = lax.axis_index("s")
        wid = s * NC + c
        row0 = s * rows_per_tile
        pltpu.sync_copy(x_hbm.at[pl.ds(row0, rows_per_tile)],
                        acc_sh.at[pl.ds(row0, rows_per_tile)])
        if tail_rows:
            @pl.when(s == 0)
            def _seed_tail():
                pltpu.sync_copy(x_hbm.at[pl.ds(tail_base, tail_rows)],
                                acc_sh.at[pl.ds(tail_base, tail_rows)])
        plsc.subcore_barrier()

        def block(blk, carry):
            pltpu.sync_copy(src_hbm.at[wid, blk], src_v)
            pltpu.sync_copy(dst_hbm.at[wid, blk], dst_v)

            for b in range(NBUF):
                pltpu.async_copy(x_hbm.at[src_v.at[b]], rows_v.at[b], sem.at[b])

            def body(i, carry2):
                b = lax.rem(i, NBUF)
                pltpu.make_async_copy(x_hbm.at[src_v.at[i]], rows_v.at[b],
                                      sem.at[b]).wait()
                pltpu.sync_copy(rows_v.at[b], acc_sh.at[dst_v.at[i]], add=True)
                nxt = i + NBUF

                @pl.when(nxt < SB)
                def _start_next():
                    pltpu.async_copy(x_hbm.at[src_v.at[nxt]], rows_v.at[b],
                                     sem.at[b])
                return carry2

            lax.fori_loop(0, SB, body, 0)
            return carry

        lax.fori_loop(0, NBLK, block, 0)
        plsc.subcore_barrier()
        pltpu.sync_copy(acc_sh.at[pl.ds(row0, rows_per_tile)],
                        out_hbm.at[c, pl.ds(row0, rows_per_tile)])
        if tail_rows:
            @pl.when(s == 0)
            def _write_tail():
                pltpu.sync_copy(acc_sh.at[pl.ds(tail_base, tail_rows)],
                                out_hbm.at[c, pl.ds(tail_base, tail_rows)])

    return agg


def _dense_layer(h0, W1, b1, g1, be1, W2, b2, g, be):
    def bn(h, gg, bb):
        m = jnp.mean(h, axis=0, keepdims=True)
        v = jnp.mean(h * h, axis=0, keepdims=True) - m * m
        return (h - m) * lax.rsqrt(v + 1e-5) * gg + bb

    h = jnp.dot(h0, W1, preferred_element_type=jnp.float32) + b1
    h = jnp.maximum(bn(h, g1, be1), 0.0)
    h = jnp.dot(h, W2, preferred_element_type=jnp.float32) + b2
    return jnp.maximum(bn(h, g, be), 0.0)


def _tc_layer0_body(x_ref, p_ref, W1_ref, b1_ref, g1_ref, be1_ref,
                    W2_ref, b2_ref, g_ref, be_ref, out_ref):
    h0 = p_ref[0] + p_ref[1] - x_ref[...]
    out_ref[...] = _dense_layer(h0, W1_ref[...], b1_ref[...], g1_ref[...],
                                be1_ref[...], W2_ref[...], b2_ref[...],
                                g_ref[...], be_ref[...])


def _tc_final_body(h1_ref, p_ref, batch_ref, W1_ref, b1_ref, g1_ref, be1_ref,
                   W2_ref, b2_ref, g_ref, be_ref, fc1W_ref, fc1b_ref,
                   fc2W_ref, fc2b_ref, out_ref, *, num_graphs):
    h1 = h1_ref[...]
    h0 = p_ref[0] + p_ref[1] - h1
    h2 = _dense_layer(h0, W1_ref[...], b1_ref[...], g1_ref[...], be1_ref[...],
                      W2_ref[...], b2_ref[...], g_ref[...], be_ref[...])
    n = h1.shape[0]
    gids = lax.broadcasted_iota(jnp.int32, (num_graphs, n), 0)
    onehot = (batch_ref[...] == gids).astype(jnp.float32)
    pooled0 = jnp.dot(onehot, h1, preferred_element_type=jnp.float32)
    pooled1 = jnp.dot(onehot, h2, preferred_element_type=jnp.float32)
    cat = jnp.concatenate([pooled0, pooled1], axis=-1)
    o = jnp.maximum(jnp.dot(cat, fc1W_ref[...],
                            preferred_element_type=jnp.float32) + fc1b_ref[...], 0.0)
    out_ref[...] = jnp.dot(o, fc2W_ref[...],
                           preferred_element_type=jnp.float32) + fc2b_ref[...]


def kernel(x, edge_index, batch,
           W1_0, b1_0, g1_0, be1_0, W2_0, b2_0, g_0, be_0,
           W1_1, b1_1, g1_1, be1_1, W2_1, b2_1, g_1, be_1,
           fc1_W, fc1_b, fc2_W, fc2_b):
    n, d = x.shape
    e = edge_index.shape[1]
    h_dim = W1_0.shape[1]
    num_graphs = 64
    c_dim = fc2_W.shape[1]

    nw = NC * NS
    src2 = edge_index[0].reshape(nw, NBLK, SB, CH)
    dst2 = edge_index[1].reshape(nw, NBLK, SB, CH)
    batch2 = batch.reshape(1, n)

    agg_fn = _make_agg_kernel(n, d, e)

    p0 = agg_fn(x, src2, dst2)
    h1 = pl.pallas_call(
        _tc_layer0_body,
        out_shape=jax.ShapeDtypeStruct((n, h_dim), jnp.float32),
    )(x, p0, W1_0, b1_0.reshape(1, -1), g1_0.reshape(1, -1),
      be1_0.reshape(1, -1), W2_0, b2_0.reshape(1, -1), g_0.reshape(1, -1),
      be_0.reshape(1, -1))

    p1 = agg_fn(h1, src2, dst2)
    out = pl.pallas_call(
        functools.partial(_tc_final_body, num_graphs=num_graphs),
        out_shape=jax.ShapeDtypeStruct((num_graphs, c_dim), jnp.float32),
    )(h1, p1, batch2, W1_1, b1_1.reshape(1, -1), g1_1.reshape(1, -1),
      be1_1.reshape(1, -1), W2_1, b2_1.reshape(1, -1), g_1.reshape(1, -1),
      be_1.reshape(1, -1), fc1_W, fc1_b.reshape(1, -1), fc2_W,
      fc2_b.reshape(1, -1))
    return out


# speedup vs baseline: 4.1094x; 1.0005x over previous
import functools

import jax
import jax.numpy as jnp
from jax import lax
from jax.experimental import pallas as pl
from jax.experimental.pallas import tpu as pltpu
from jax.experimental.pallas import tpu_sc as plsc

NC = 2
NS = 16
CH = 80
NBUF = 4
SB = 25
NBLK = 5


def _make_agg_kernel(n, d, e):
    nw = NC * NS
    assert e == nw * NBLK * SB * CH
    rows_per_tile = (n // NS) // 8 * 8
    tail_rows = n - rows_per_tile * NS
    tail_base = rows_per_tile * NS
    assert tail_base % 8 == 0 and tail_rows % 8 == 0

    mesh = plsc.VectorSubcoreMesh(core_axis_name="c", subcore_axis_name="s")

    @functools.partial(
        pl.kernel,
        out_type=jax.ShapeDtypeStruct((NC, n, d), jnp.float32),
        mesh=mesh,
        scratch_types=[
            pltpu.VMEM((SB, CH), jnp.int32),
            pltpu.VMEM((SB, CH), jnp.int32),
            pltpu.VMEM((NBUF, CH, d), jnp.float32),
            pltpu.VMEM_SHARED((n, d), jnp.float32),
            pltpu.SemaphoreType.DMA((NBUF,)),
        ],
    )
    def agg(x_hbm, src_hbm, dst_hbm, out_hbm, src_v, dst_v, rows_v, acc_sh, sem):
        c = lax.axis_index("c")
        s = lax.axis_index("s")
        wid = s * NC + c
        row0 = s * rows_per_tile
        pltpu.sync_copy(x_hbm.at[pl.ds(row0, rows_per_tile)],
                        acc_sh.at[pl.ds(row0, rows_per_tile)])
        if tail_rows:
            @pl.when(s == 0)
            def _seed_tail():
                pltpu.sync_copy(x_hbm.at[pl.ds(tail_base, tail_rows)],
                                acc_sh.at[pl.ds(tail_base, tail_rows)])
        plsc.subcore_barrier()

        def block(blk, carry):
            pltpu.sync_copy(src_hbm.at[wid, blk], src_v)
            pltpu.sync_copy(dst_hbm.at[wid, blk], dst_v)

            for b in range(NBUF):
                pltpu.async_copy(x_hbm.at[src_v.at[b]], rows_v.at[b], sem.at[b])

            def body(i, carry2):
                b = lax.rem(i, NBUF)
                pltpu.make_async_copy(x_hbm.at[src_v.at[i]], rows_v.at[b],
                                      sem.at[b]).wait()
                pltpu.sync_copy(rows_v.at[b], acc_sh.at[dst_v.at[i]], add=True)
                nxt = i + NBUF

                @pl.when(nxt < SB)
                def _start_next():
                    pltpu.async_copy(x_hbm.at[src_v.at[nxt]], rows_v.at[b],
                                     sem.at[b])
                return carry2

            lax.fori_loop(0, SB, body, 0)
            return carry

        lax.fori_loop(0, NBLK, block, 0)
        plsc.subcore_barrier()
        pltpu.sync_copy(acc_sh.at[pl.ds(row0, rows_per_tile)],
                        out_hbm.at[c, pl.ds(row0, rows_per_tile)])
        if tail_rows:
            @pl.when(s == 0)
            def _write_tail():
                pltpu.sync_copy(acc_sh.at[pl.ds(tail_base, tail_rows)],
                                out_hbm.at[c, pl.ds(tail_base, tail_rows)])

    return agg


def _dense_layer(h0, W1, b1, g1, be1, W2, b2, g, be):
    def bn(h, gg, bb):
        m = jnp.mean(h, axis=0, keepdims=True)
        v = jnp.mean(h * h, axis=0, keepdims=True) - m * m
        return (h - m) * lax.rsqrt(v + 1e-5) * gg + bb

    h = jnp.dot(h0, W1, preferred_element_type=jnp.float32) + b1
    h = jnp.maximum(bn(h, g1, be1), 0.0)
    h = jnp.dot(h, W2, preferred_element_type=jnp.float32) + b2
    return jnp.maximum(bn(h, g, be), 0.0)


def _tc_layer0_body(x_ref, p_ref, W1_ref, b1_ref, g1_ref, be1_ref,
                    W2_ref, b2_ref, g_ref, be_ref, out_ref):
    h0 = p_ref[0] + p_ref[1] - x_ref[...]
    out_ref[...] = _dense_layer(h0, W1_ref[...], b1_ref[...], g1_ref[...],
                                be1_ref[...], W2_ref[...], b2_ref[...],
                                g_ref[...], be_ref[...])


def _tc_final_body(h1_ref, p_ref, batch_ref, W1_ref, b1_ref, g1_ref, be1_ref,
                   W2_ref, b2_ref, g_ref, be_ref, fc1W_ref, fc1b_ref,
                   fc2W_ref, fc2b_ref, out_ref, *, num_graphs):
    h1 = h1_ref[...]
    h0 = p_ref[0] + p_ref[1] - h1
    h2 = _dense_layer(h0, W1_ref[...], b1_ref[...], g1_ref[...], be1_ref[...],
                      W2_ref[...], b2_ref[...], g_ref[...], be_ref[...])
    n = h1.shape[0]
    gids = lax.broadcasted_iota(jnp.int32, (num_graphs, n), 0)
    onehot = (batch_ref[...] == gids).astype(jnp.float32)
    cat = jnp.dot(onehot, jnp.concatenate([h1, h2], axis=-1),
                  preferred_element_type=jnp.float32)
    o = jnp.maximum(jnp.dot(cat, fc1W_ref[...],
                            preferred_element_type=jnp.float32) + fc1b_ref[...], 0.0)
    out_ref[...] = jnp.dot(o, fc2W_ref[...],
                           preferred_element_type=jnp.float32) + fc2b_ref[...]


def kernel(x, edge_index, batch,
           W1_0, b1_0, g1_0, be1_0, W2_0, b2_0, g_0, be_0,
           W1_1, b1_1, g1_1, be1_1, W2_1, b2_1, g_1, be_1,
           fc1_W, fc1_b, fc2_W, fc2_b):
    n, d = x.shape
    e = edge_index.shape[1]
    h_dim = W1_0.shape[1]
    num_graphs = 64
    c_dim = fc2_W.shape[1]

    nw = NC * NS
    src2 = edge_index[0].reshape(nw, NBLK, SB, CH)
    dst2 = edge_index[1].reshape(nw, NBLK, SB, CH)
    batch2 = batch.reshape(1, n)

    agg_fn = _make_agg_kernel(n, d, e)

    p0 = agg_fn(x, src2, dst2)
    h1 = pl.pallas_call(
        _tc_layer0_body,
        out_shape=jax.ShapeDtypeStruct((n, h_dim), jnp.float32),
    )(x, p0, W1_0, b1_0.reshape(1, -1), g1_0.reshape(1, -1),
      be1_0.reshape(1, -1), W2_0, b2_0.reshape(1, -1), g_0.reshape(1, -1),
      be_0.reshape(1, -1))

    p1 = agg_fn(h1, src2, dst2)
    out = pl.pallas_call(
        functools.partial(_tc_final_body, num_graphs=num_graphs),
        out_shape=jax.ShapeDtypeStruct((num_graphs, c_dim), jnp.float32),
    )(h1, p1, batch2, W1_1, b1_1.reshape(1, -1), g1_1.reshape(1, -1),
      be1_1.reshape(1, -1), W2_1, b2_1.reshape(1, -1), g_1.reshape(1, -1),
      be_1.reshape(1, -1), fc1_W, fc1_b.reshape(1, -1), fc2_W,
      fc2_b.reshape(1, -1))
    return out


# speedup vs baseline: 4.1105x; 1.0003x over previous
import functools

import jax
import jax.numpy as jnp
from jax import lax
from jax.experimental import pallas as pl
from jax.experimental.pallas import tpu as pltpu
from jax.experimental.pallas import tpu_sc as plsc

NC = 2
NS = 16
CH = 80
NBUF = 4
SB = 25
NBLK = 5


def _make_agg_kernel(n, d, e):
    nw = NC * NS
    assert e == nw * NBLK * SB * CH
    rows_per_tile = (n // NS) // 8 * 8
    tail_rows = n - rows_per_tile * NS
    tail_base = rows_per_tile * NS
    assert tail_base % 8 == 0 and tail_rows % 8 == 0

    mesh = plsc.VectorSubcoreMesh(core_axis_name="c", subcore_axis_name="s")

    @functools.partial(
        pl.kernel,
        out_type=jax.ShapeDtypeStruct((NC, n, d), jnp.float32),
        mesh=mesh,
        scratch_types=[
            pltpu.VMEM((SB, CH), jnp.int32),
            pltpu.VMEM((SB, CH), jnp.int32),
            pltpu.VMEM((NBUF, CH, d), jnp.float32),
            pltpu.VMEM_SHARED((n, d), jnp.float32),
            pltpu.SemaphoreType.DMA((NBUF,)),
        ],
    )
    def agg(x_hbm, src_hbm, dst_hbm, out_hbm, src_v, dst_v, rows_v, acc_sh, sem):
        c = lax.axis_index("c")
        s = lax.axis_index("s")
        wid = s * NC + c
        row0 = s * rows_per_tile
        pltpu.sync_copy(x_hbm.at[pl.ds(row0, rows_per_tile)],
                        acc_sh.at[pl.ds(row0, rows_per_tile)])
        if tail_rows:
            @pl.when(s == 0)
            def _seed_tail():
                pltpu.sync_copy(x_hbm.at[pl.ds(tail_base, tail_rows)],
                                acc_sh.at[pl.ds(tail_base, tail_rows)])
        plsc.subcore_barrier()

        def block(blk, carry):
            pltpu.sync_copy(src_hbm.at[wid, blk], src_v)
            pltpu.sync_copy(dst_hbm.at[wid, blk], dst_v)

            for b in range(NBUF):
                pltpu.async_copy(x_hbm.at[src_v.at[b]], rows_v.at[b], sem.at[b])

            def body(i, carry2):
                b = lax.bitwise_and(i, NBUF - 1)
                pltpu.make_async_copy(x_hbm.at[src_v.at[i]], rows_v.at[b],
                                      sem.at[b]).wait()
                pltpu.sync_copy(rows_v.at[b], acc_sh.at[dst_v.at[i]], add=True)
                nxt = i + NBUF

                @pl.when(nxt < SB)
                def _start_next():
                    pltpu.async_copy(x_hbm.at[src_v.at[nxt]], rows_v.at[b],
                                     sem.at[b])
                return carry2

            lax.fori_loop(0, SB, body, 0)
            return carry

        lax.fori_loop(0, NBLK, block, 0)
        plsc.subcore_barrier()
        pltpu.sync_copy(acc_sh.at[pl.ds(row0, rows_per_tile)],
                        out_hbm.at[c, pl.ds(row0, rows_per_tile)])
        if tail_rows:
            @pl.when(s == 0)
            def _write_tail():
                pltpu.sync_copy(acc_sh.at[pl.ds(tail_base, tail_rows)],
                                out_hbm.at[c, pl.ds(tail_base, tail_rows)])

    return agg


def _dense_layer(h0, W1, b1, g1, be1, W2, b2, g, be):
    def bn(h, gg, bb):
        m = jnp.mean(h, axis=0, keepdims=True)
        v = jnp.mean(h * h, axis=0, keepdims=True) - m * m
        return (h - m) * lax.rsqrt(v + 1e-5) * gg + bb

    h = jnp.dot(h0, W1, preferred_element_type=jnp.float32) + b1
    h = jnp.maximum(bn(h, g1, be1), 0.0)
    h = jnp.dot(h, W2, preferred_element_type=jnp.float32) + b2
    return jnp.maximum(bn(h, g, be), 0.0)


def _tc_layer0_body(x_ref, p_ref, W1_ref, b1_ref, g1_ref, be1_ref,
                    W2_ref, b2_ref, g_ref, be_ref, out_ref):
    h0 = p_ref[0] + p_ref[1] - x_ref[...]
    out_ref[...] = _dense_layer(h0, W1_ref[...], b1_ref[...], g1_ref[...],
                                be1_ref[...], W2_ref[...], b2_ref[...],
                                g_ref[...], be_ref[...])


def _tc_final_body(h1_ref, p_ref, batch_ref, W1_ref, b1_ref, g1_ref, be1_ref,
                   W2_ref, b2_ref, g_ref, be_ref, fc1W_ref, fc1b_ref,
                   fc2W_ref, fc2b_ref, out_ref, *, num_graphs):
    h1 = h1_ref[...]
    h0 = p_ref[0] + p_ref[1] - h1
    h2 = _dense_layer(h0, W1_ref[...], b1_ref[...], g1_ref[...], be1_ref[...],
                      W2_ref[...], b2_ref[...], g_ref[...], be_ref[...])
    n = h1.shape[0]
    gids = lax.broadcasted_iota(jnp.int32, (num_graphs, n), 0)
    onehot = (batch_ref[...] == gids).astype(jnp.float32)
    cat = jnp.dot(onehot, jnp.concatenate([h1, h2], axis=-1),
                  preferred_element_type=jnp.float32)
    o = jnp.maximum(jnp.dot(cat, fc1W_ref[...],
                            preferred_element_type=jnp.float32) + fc1b_ref[...], 0.0)
    out_ref[...] = jnp.dot(o, fc2W_ref[...],
                           preferred_element_type=jnp.float32) + fc2b_ref[...]


def kernel(x, edge_index, batch,
           W1_0, b1_0, g1_0, be1_0, W2_0, b2_0, g_0, be_0,
           W1_1, b1_1, g1_1, be1_1, W2_1, b2_1, g_1, be_1,
           fc1_W, fc1_b, fc2_W, fc2_b):
    n, d = x.shape
    e = edge_index.shape[1]
    h_dim = W1_0.shape[1]
    num_graphs = 64
    c_dim = fc2_W.shape[1]

    nw = NC * NS
    src2 = edge_index[0].reshape(nw, NBLK, SB, CH)
    dst2 = edge_index[1].reshape(nw, NBLK, SB, CH)
    batch2 = batch.reshape(1, n)

    agg_fn = _make_agg_kernel(n, d, e)

    p0 = agg_fn(x, src2, dst2)
    h1 = pl.pallas_call(
        _tc_layer0_body,
        out_shape=jax.ShapeDtypeStruct((n, h_dim), jnp.float32),
    )(x, p0, W1_0, b1_0.reshape(1, -1), g1_0.reshape(1, -1),
      be1_0.reshape(1, -1), W2_0, b2_0.reshape(1, -1), g_0.reshape(1, -1),
      be_0.reshape(1, -1))

    p1 = agg_fn(h1, src2, dst2)
    out = pl.pallas_call(
        functools.partial(_tc_final_body, num_graphs=num_graphs),
        out_shape=jax.ShapeDtypeStruct((num_graphs, c_dim), jnp.float32),
    )(h1, p1, batch2, W1_1, b1_1.reshape(1, -1), g1_1.reshape(1, -1),
      be1_1.reshape(1, -1), W2_1, b2_1.reshape(1, -1), g_1.reshape(1, -1),
      be_1.reshape(1, -1), fc1_W, fc1_b.reshape(1, -1), fc2_W,
      fc2_b.reshape(1, -1))
    return out
